# baseline (speedup 1.0000x reference)
import functools

import jax
import jax.numpy as jnp
from jax import lax
from jax.experimental import pallas as pl
from jax.experimental.pallas import tpu as pltpu
from jax.experimental.pallas import tpu_sc as plsc

D = 4096
E = 64
K = 8
N = 8192
T = 512
G = N // T


def _tc_body(x_ref, wt_ref, w_out, k_out, l_out, base_out, laux_out,
             me_s, ce_s, carry_s):
    pid = pl.program_id(0)

    @pl.when(pid == 0)
    def _():
        me_s[...] = jnp.zeros_like(me_s)
        ce_s[...] = jnp.zeros_like(ce_s)
        carry_s[...] = jnp.zeros_like(carry_s)

    xb = x_ref[...]
    logits = jnp.dot(xb, wt_ref[...], preferred_element_type=jnp.float32)
    mx = jnp.max(logits, axis=1, keepdims=True)
    ex = jnp.exp(logits - mx)
    sm = jnp.sum(ex, axis=1, keepdims=True)
    gates = ex / sm
    me_s[...] += jnp.sum(gates, axis=0, keepdims=True)

    col = lax.broadcasted_iota(jnp.int32, (T, E), 1)
    g = gates
    wcols = []
    icols = []
    onehots = []
    for _ in range(K):
        mj = jnp.max(g, axis=1, keepdims=True)
        cand = jnp.where(g == mj, col, E)
        selj = jnp.min(cand, axis=1, keepdims=True)
        oh = col == selj
        wcols.append(mj)
        icols.append(selj)
        onehots.append(oh)
        g = jnp.where(oh, -jnp.inf, g)

    w_out[...] = jnp.concatenate(wcols, axis=1)
    k_out[...] = jnp.concatenate(icols, axis=1)
    ce_s[...] += jnp.sum(onehots[0].astype(jnp.float32), axis=0, keepdims=True)

    m_sel = onehots[0].astype(jnp.float32)
    for j in range(1, K):
        m_sel = m_sel + onehots[j].astype(jnp.float32)

    ri = lax.broadcasted_iota(jnp.int32, (T, T), 0)
    ci = lax.broadcasted_iota(jnp.int32, (T, T), 1)
    lt = (ci < ri).astype(jnp.float32)
    csum = lax.dot(lt, m_sel, precision=lax.Precision.HIGHEST) + carry_s[...]
    lcols = [
        jnp.sum(jnp.where(onehots[j], csum, 0.0), axis=1, keepdims=True)
        for j in range(K)
    ]
    l_out[...] = jnp.concatenate(lcols, axis=1).astype(jnp.int32)
    carry_s[...] += jnp.sum(m_sel, axis=0, keepdims=True)

    @pl.when(pid == G - 1)
    def _():
        counts = carry_s[...]
        er = lax.broadcasted_iota(jnp.int32, (E, E), 0)
        ec = lax.broadcasted_iota(jnp.int32, (E, E), 1)
        ut = (er < ec).astype(jnp.float32)
        base_out[...] = lax.dot(
            counts, ut, precision=lax.Precision.HIGHEST).astype(jnp.int32)
        laux_out[...] = jnp.sum(
            me_s[...] * ce_s[...], keepdims=True) * (float(E) / (float(N) * float(N)))


def _tc_call(x, wt):
    return pl.pallas_call(
        _tc_body,
        grid=(G,),
        in_specs=[
            pl.BlockSpec((T, D), lambda i: (i, 0)),
            pl.BlockSpec((D, E), lambda i: (0, 0)),
        ],
        out_specs=[
            pl.BlockSpec((T, K), lambda i: (i, 0)),
            pl.BlockSpec((T, K), lambda i: (i, 0)),
            pl.BlockSpec((T, K), lambda i: (i, 0)),
            pl.BlockSpec((1, E), lambda i: (0, 0)),
            pl.BlockSpec((1, 1), lambda i: (0, 0)),
        ],
        out_shape=[
            jax.ShapeDtypeStruct((N, K), jnp.float32),
            jax.ShapeDtypeStruct((N, K), jnp.int32),
            jax.ShapeDtypeStruct((N, K), jnp.int32),
            jax.ShapeDtypeStruct((1, E), jnp.int32),
            jax.ShapeDtypeStruct((1, 1), jnp.float32),
        ],
        scratch_shapes=[
            pltpu.VMEM((1, E), jnp.float32),
            pltpu.VMEM((1, E), jnp.float32),
            pltpu.VMEM((1, E), jnp.float32),
        ],
    )(x, wt)


_NW = 32
_CHUNK = (N * K) // _NW
_ROWS = 16
_BATCH = 128


def _sc_scatter_body(eid_hbm, lsel_hbm, base_hbm, tok_out, eid_out,
                     eid_v, lsel_v, base_v, pos2d, tok2d, eid2d, sem):
    wid = lax.axis_index("s") * 2 + lax.axis_index("c")
    start = wid * _CHUNK
    pltpu.sync_copy(eid_hbm.at[pl.ds(start, _CHUNK)], eid_v)
    pltpu.sync_copy(lsel_hbm.at[pl.ds(start, _CHUNK)], lsel_v)
    pltpu.sync_copy(base_hbm, base_v)

    lane = lax.iota(jnp.int32, 16)
    for r in range(_ROWS):
        for c in range(_BATCH // 16):
            off = r * _BATCH + c * 16
            ev = eid_v[pl.ds(off, 16)]
            lv = lsel_v[pl.ds(off, 16)]
            bv = plsc.load_gather(base_v, [ev])
            pos = bv + lv
            tok = lax.shift_right_logical(start + off + lane, 3)
            pos2d[r, pl.ds(c * 16, 16)] = pos
            tok2d[r, pl.ds(c * 16, 16)] = tok
            eid2d[r, pl.ds(c * 16, 16)] = ev

    copies = []
    for r in range(_ROWS):
        copies.append(pltpu.async_copy(tok2d.at[r], tok_out.at[pos2d.at[r]], sem))
    for r in range(_ROWS):
        copies.append(pltpu.async_copy(eid2d.at[r], eid_out.at[pos2d.at[r]], sem))
    for cp in copies:
        cp.wait()


def _sc_scatter(eid_flat, lsel_flat, base):
    mesh = plsc.VectorSubcoreMesh(core_axis_name="c", subcore_axis_name="s")
    kern = pl.kernel(
        _sc_scatter_body,
        mesh=mesh,
        out_type=[
            jax.ShapeDtypeStruct((N * K,), jnp.int32),
            jax.ShapeDtypeStruct((N * K,), jnp.int32),
        ],
        scratch_types=[
            pltpu.VMEM((_CHUNK,), jnp.int32),
            pltpu.VMEM((_CHUNK,), jnp.int32),
            pltpu.VMEM((E,), jnp.int32),
            pltpu.VMEM((_ROWS, _BATCH), jnp.int32),
            pltpu.VMEM((_ROWS, _BATCH), jnp.int32),
            pltpu.VMEM((_ROWS, _BATCH), jnp.int32),
            pltpu.SemaphoreType.DMA,
        ],
        compiler_params=pltpu.CompilerParams(needs_layout_passes=False),
    )
    return kern(eid_flat, lsel_flat, base)


def kernel(x, W):
    wt = W.T
    weights, topk, lsel, base, laux = _tc_call(x, wt)
    eid_flat = topk.reshape(-1)
    lsel_flat = lsel.reshape(-1)
    indices, expert_ids = _sc_scatter(eid_flat, lsel_flat, base.reshape(-1))
    l_aux = laux[0, 0]
    return l_aux, weights, indices, expert_ids


# speedup vs baseline: 1.0082x; 1.0082x over previous
import functools

import jax
import jax.numpy as jnp
from jax import lax
from jax.experimental import pallas as pl
from jax.experimental.pallas import tpu as pltpu
from jax.experimental.pallas import tpu_sc as plsc

D = 4096
E = 64
K = 8
N = 8192
T = 512
G = N // T


def _tc_body(x_ref, wt_ref, w_out, k_out, l_out, base_out, laux_out,
             me_s, ce_s, carry_s):
    pid = pl.program_id(0)

    @pl.when(pid == 0)
    def _():
        me_s[...] = jnp.zeros_like(me_s)
        ce_s[...] = jnp.zeros_like(ce_s)
        carry_s[...] = jnp.zeros_like(carry_s)

    xb = x_ref[...]
    logits = jnp.dot(xb, wt_ref[...], preferred_element_type=jnp.float32)
    mx = jnp.max(logits, axis=1, keepdims=True)
    ex = jnp.exp(logits - mx)
    sm = jnp.sum(ex, axis=1, keepdims=True)
    gates = ex / sm
    me_s[...] += jnp.sum(gates, axis=0, keepdims=True)

    col = lax.broadcasted_iota(jnp.int32, (T, E), 1)
    g = gates
    wcols = []
    icols = []
    onehots = []
    for _ in range(K):
        mj = jnp.max(g, axis=1, keepdims=True)
        cand = jnp.where(g == mj, col, E)
        selj = jnp.min(cand, axis=1, keepdims=True)
        oh = col == selj
        wcols.append(mj)
        icols.append(selj)
        onehots.append(oh)
        g = jnp.where(oh, -jnp.inf, g)

    w_out[...] = jnp.concatenate(wcols, axis=1)
    k_out[...] = jnp.concatenate(icols, axis=1)
    ce_s[...] += jnp.sum(onehots[0].astype(jnp.float32), axis=0, keepdims=True)

    m_sel = onehots[0].astype(jnp.float32)
    for j in range(1, K):
        m_sel = m_sel + onehots[j].astype(jnp.float32)

    ri = lax.broadcasted_iota(jnp.int32, (T, T), 0)
    ci = lax.broadcasted_iota(jnp.int32, (T, T), 1)
    lt = (ci < ri).astype(jnp.float32)
    csum = lax.dot(lt, m_sel, precision=lax.Precision.HIGHEST) + carry_s[...]
    lcols = [
        jnp.sum(jnp.where(onehots[j], csum, 0.0), axis=1, keepdims=True)
        for j in range(K)
    ]
    l_out[...] = jnp.concatenate(lcols, axis=1).astype(jnp.int32)
    carry_s[...] += jnp.sum(m_sel, axis=0, keepdims=True)

    @pl.when(pid == G - 1)
    def _():
        counts = carry_s[...]
        er = lax.broadcasted_iota(jnp.int32, (E, E), 0)
        ec = lax.broadcasted_iota(jnp.int32, (E, E), 1)
        ut = (er < ec).astype(jnp.float32)
        base_out[...] = lax.dot(
            counts, ut, precision=lax.Precision.HIGHEST).astype(jnp.int32)
        laux_out[...] = jnp.sum(
            me_s[...] * ce_s[...], keepdims=True) * (float(E) / (float(N) * float(N)))


def _tc_call(x, wt):
    return pl.pallas_call(
        _tc_body,
        grid=(G,),
        in_specs=[
            pl.BlockSpec((T, D), lambda i: (i, 0)),
            pl.BlockSpec((D, E), lambda i: (0, 0)),
        ],
        out_specs=[
            pl.BlockSpec((T, K), lambda i: (i, 0)),
            pl.BlockSpec((T, K), lambda i: (i, 0)),
            pl.BlockSpec((T, K), lambda i: (i, 0)),
            pl.BlockSpec((1, E), lambda i: (0, 0)),
            pl.BlockSpec((1, 1), lambda i: (0, 0)),
        ],
        out_shape=[
            jax.ShapeDtypeStruct((N, K), jnp.float32),
            jax.ShapeDtypeStruct((N, K), jnp.int32),
            jax.ShapeDtypeStruct((N, K), jnp.int32),
            jax.ShapeDtypeStruct((1, E), jnp.int32),
            jax.ShapeDtypeStruct((1, 1), jnp.float32),
        ],
        scratch_shapes=[
            pltpu.VMEM((1, E), jnp.float32),
            pltpu.VMEM((1, E), jnp.float32),
            pltpu.VMEM((1, E), jnp.float32),
        ],
    )(x, wt)


_NW = 32
_CHUNK = (N * K) // _NW
_ROWS = 16
_BATCH = 128


def _sc_scatter_body(eid_hbm, lsel_hbm, base_hbm, tok_out, eid_out,
                     eid_v, lsel_v, base_v, pos_v, tok_v, sem):
    wid = lax.axis_index("s") * 2 + lax.axis_index("c")
    start = wid * _CHUNK
    pltpu.sync_copy(eid_hbm.at[pl.ds(start, _CHUNK)], eid_v)
    pltpu.sync_copy(lsel_hbm.at[pl.ds(start, _CHUNK)], lsel_v)
    pltpu.sync_copy(base_hbm, base_v)

    lane = lax.iota(jnp.int32, 16)
    for v in range(_CHUNK // 16):
        off = v * 16
        ev = eid_v[pl.ds(off, 16)]
        lv = lsel_v[pl.ds(off, 16)]
        bv = plsc.load_gather(base_v, [ev])
        pos_v[pl.ds(off, 16)] = bv + lv
        tok_v[pl.ds(off, 16)] = lax.shift_right_logical(start + off + lane, 3)

    c1 = pltpu.async_copy(tok_v, tok_out.at[pos_v], sem)
    c2 = pltpu.async_copy(eid_v, eid_out.at[pos_v], sem)
    c1.wait()
    c2.wait()


def _sc_scatter(eid_flat, lsel_flat, base):
    mesh = plsc.VectorSubcoreMesh(core_axis_name="c", subcore_axis_name="s")
    kern = pl.kernel(
        _sc_scatter_body,
        mesh=mesh,
        out_type=[
            jax.ShapeDtypeStruct((N * K,), jnp.int32),
            jax.ShapeDtypeStruct((N * K,), jnp.int32),
        ],
        scratch_types=[
            pltpu.VMEM((_CHUNK,), jnp.int32),
            pltpu.VMEM((_CHUNK,), jnp.int32),
            pltpu.VMEM((E,), jnp.int32),
            pltpu.VMEM((_CHUNK,), jnp.int32),
            pltpu.VMEM((_CHUNK,), jnp.int32),
            pltpu.SemaphoreType.DMA,
        ],
        compiler_params=pltpu.CompilerParams(needs_layout_passes=False),
    )
    return kern(eid_flat, lsel_flat, base)


def kernel(x, W):
    wt = W.T
    weights, topk, lsel, base, laux = _tc_call(x, wt)
    eid_flat = topk.reshape(-1)
    lsel_flat = lsel.reshape(-1)
    indices, expert_ids = _sc_scatter(eid_flat, lsel_flat, base.reshape(-1))
    l_aux = laux[0, 0]
    return l_aux, weights, indices, expert_ids
